# baseline (speedup 1.0000x reference)
import jax
import jax.numpy as jnp
from jax import lax
from jax.experimental import pallas as pl
from jax.experimental.pallas import tpu as pltpu
from jax.experimental.pallas import tpu_sc as plsc

_N, _C, _L = 32, 128, 8192
_LANES = 16
_CHUNKS = _L // _LANES
_NBLK = _C // 4


def _sc_body(inp_hbm, grid_hbm, out_hbm, grid_v, idx_v, w1_v,
             in_a0, in_a1, in_b0, in_b1, out_a0, out_a1, out_b0, out_b1,
             si_a, si_b, so_a, so_b):
    core = lax.axis_index("c")
    sub = lax.axis_index("s")
    w = sub * 2 + core

    pltpu.sync_copy(grid_hbm.at[w], grid_v)
    pltpu.async_copy(inp_hbm.at[w, 0], in_a0, si_a)
    pltpu.async_copy(inp_hbm.at[w, 1], in_a1, si_a)

    def _widx(k, carry):
        s = pl.ds(k * _LANES, _LANES)
        g = grid_v[s]
        ix = (g + 1.0) * (0.5 * (_L - 1))
        ix = jnp.minimum(jnp.maximum(ix, 0.0), float(_L - 1))
        i0 = ix.astype(jnp.int32)
        idx_v[s] = i0
        w1_v[s] = ix - i0.astype(jnp.float32)
        return carry

    lax.fori_loop(0, _CHUNKS, _widx, 0, unroll=2)

    def _compute(in0, in1, out0, out1):
        def _chunk(k, inner):
            s = pl.ds(k * _LANES, _LANES)
            i0 = idx_v[s]
            w1 = w1_v[s]
            i1 = jnp.minimum(i0 + 1, _L - 1)
            v0 = plsc.load_gather(in0, [i0])
            v1 = plsc.load_gather(in0, [i1])
            out0[s] = v0 + w1 * (v1 - v0)
            u0 = plsc.load_gather(in1, [i0])
            u1 = plsc.load_gather(in1, [i1])
            out1[s] = u0 + w1 * (u1 - u0)
            return inner

        lax.fori_loop(0, _CHUNKS, _chunk, 0, unroll=2)

    def _loop(j, carry):
        c0 = 4 * j
        pltpu.async_copy(inp_hbm.at[w, c0 + 2], in_b0, si_b)
        pltpu.async_copy(inp_hbm.at[w, c0 + 3], in_b1, si_b)
        pltpu.make_async_copy(inp_hbm.at[w, c0], in_a0, si_a).wait()
        pltpu.make_async_copy(inp_hbm.at[w, c0 + 1], in_a1, si_a).wait()

        @pl.when(j > 0)
        def _():
            pltpu.make_async_copy(out_a0, out_hbm.at[w, c0], so_a).wait()
            pltpu.make_async_copy(out_a1, out_hbm.at[w, c0], so_a).wait()

        _compute(in_a0, in_a1, out_a0, out_a1)
        pltpu.async_copy(out_a0, out_hbm.at[w, c0], so_a)
        pltpu.async_copy(out_a1, out_hbm.at[w, c0 + 1], so_a)

        @pl.when(j < _NBLK - 1)
        def _():
            pltpu.async_copy(inp_hbm.at[w, c0 + 4], in_a0, si_a)
            pltpu.async_copy(inp_hbm.at[w, c0 + 5], in_a1, si_a)

        pltpu.make_async_copy(inp_hbm.at[w, c0 + 2], in_b0, si_b).wait()
        pltpu.make_async_copy(inp_hbm.at[w, c0 + 3], in_b1, si_b).wait()

        @pl.when(j > 0)
        def _():
            pltpu.make_async_copy(out_b0, out_hbm.at[w, c0], so_b).wait()
            pltpu.make_async_copy(out_b1, out_hbm.at[w, c0], so_b).wait()

        _compute(in_b0, in_b1, out_b0, out_b1)
        pltpu.async_copy(out_b0, out_hbm.at[w, c0 + 2], so_b)
        pltpu.async_copy(out_b1, out_hbm.at[w, c0 + 3], so_b)
        return carry

    lax.fori_loop(0, _NBLK, _loop, 0)

    pltpu.make_async_copy(out_a0, out_hbm.at[w, 0], so_a).wait()
    pltpu.make_async_copy(out_a1, out_hbm.at[w, 1], so_a).wait()
    pltpu.make_async_copy(out_b0, out_hbm.at[w, 2], so_b).wait()
    pltpu.make_async_copy(out_b1, out_hbm.at[w, 3], so_b).wait()


@jax.jit
def kernel(input, grid):
    mesh = plsc.VectorSubcoreMesh(core_axis_name="c", subcore_axis_name="s")
    f = pl.kernel(
        _sc_body,
        mesh=mesh,
        out_type=jax.ShapeDtypeStruct((_N, _C, _L), jnp.float32),
        compiler_params=pltpu.CompilerParams(needs_layout_passes=False),
        scratch_types=[
            pltpu.VMEM((_L,), jnp.float32),
            pltpu.VMEM((_L,), jnp.int32),
            pltpu.VMEM((_L,), jnp.float32),
            pltpu.VMEM((_L,), jnp.float32),
            pltpu.VMEM((_L,), jnp.float32),
            pltpu.VMEM((_L,), jnp.float32),
            pltpu.VMEM((_L,), jnp.float32),
            pltpu.VMEM((_L,), jnp.float32),
            pltpu.VMEM((_L,), jnp.float32),
            pltpu.VMEM((_L,), jnp.float32),
            pltpu.VMEM((_L,), jnp.float32),
            pltpu.SemaphoreType.DMA,
            pltpu.SemaphoreType.DMA,
            pltpu.SemaphoreType.DMA,
            pltpu.SemaphoreType.DMA,
        ],
    )
    return f(input, grid)


# speedup vs baseline: 4.7556x; 4.7556x over previous
import jax
import jax.numpy as jnp
from jax import lax
from jax.experimental import pallas as pl
from jax.experimental.pallas import tpu as pltpu
from jax.experimental.pallas import tpu_sc as plsc

_N, _C, _L = 32, 128, 8192
_LANES = 16
_CHUNKS = _L // _LANES
_NBLK = _C // 4


def _sc_body(inp_hbm, grid_hbm, out_hbm, grid_v, idx_v, w1_v,
             in_a0, in_a1, in_b0, in_b1, out_a0, out_a1, out_b0, out_b1,
             si_a, si_b, so_a, so_b):
    core = lax.axis_index("c")
    sub = lax.axis_index("s")
    w = sub * 2 + core

    pltpu.sync_copy(grid_hbm.at[w], grid_v)
    pltpu.async_copy(inp_hbm.at[w, 0], in_a0, si_a)
    pltpu.async_copy(inp_hbm.at[w, 1], in_a1, si_a)

    @plsc.parallel_loop(0, _CHUNKS, unroll=4)
    def _widx(k):
        s = pl.ds(k * _LANES, _LANES)
        g = grid_v[s]
        ix = (g + 1.0) * (0.5 * (_L - 1))
        ix = jnp.minimum(jnp.maximum(ix, 0.0), float(_L - 1))
        i0 = ix.astype(jnp.int32)
        idx_v[s] = i0
        w1_v[s] = ix - i0.astype(jnp.float32)

    def _compute(in0, in1, out0, out1):
        @plsc.parallel_loop(0, _CHUNKS, unroll=4)
        def _chunk(k):
            s = pl.ds(k * _LANES, _LANES)
            i0 = idx_v[s]
            w1 = w1_v[s]
            i1 = jnp.minimum(i0 + 1, _L - 1)
            v0 = plsc.load_gather(in0, [i0])
            v1 = plsc.load_gather(in0, [i1])
            out0[s] = v0 + w1 * (v1 - v0)
            u0 = plsc.load_gather(in1, [i0])
            u1 = plsc.load_gather(in1, [i1])
            out1[s] = u0 + w1 * (u1 - u0)

    def _loop(j, carry):
        c0 = 4 * j
        pltpu.async_copy(inp_hbm.at[w, c0 + 2], in_b0, si_b)
        pltpu.async_copy(inp_hbm.at[w, c0 + 3], in_b1, si_b)
        pltpu.make_async_copy(inp_hbm.at[w, c0], in_a0, si_a).wait()
        pltpu.make_async_copy(inp_hbm.at[w, c0 + 1], in_a1, si_a).wait()

        @pl.when(j > 0)
        def _():
            pltpu.make_async_copy(out_a0, out_hbm.at[w, c0], so_a).wait()
            pltpu.make_async_copy(out_a1, out_hbm.at[w, c0], so_a).wait()

        _compute(in_a0, in_a1, out_a0, out_a1)
        pltpu.async_copy(out_a0, out_hbm.at[w, c0], so_a)
        pltpu.async_copy(out_a1, out_hbm.at[w, c0 + 1], so_a)

        @pl.when(j < _NBLK - 1)
        def _():
            pltpu.async_copy(inp_hbm.at[w, c0 + 4], in_a0, si_a)
            pltpu.async_copy(inp_hbm.at[w, c0 + 5], in_a1, si_a)

        pltpu.make_async_copy(inp_hbm.at[w, c0 + 2], in_b0, si_b).wait()
        pltpu.make_async_copy(inp_hbm.at[w, c0 + 3], in_b1, si_b).wait()

        @pl.when(j > 0)
        def _():
            pltpu.make_async_copy(out_b0, out_hbm.at[w, c0], so_b).wait()
            pltpu.make_async_copy(out_b1, out_hbm.at[w, c0], so_b).wait()

        _compute(in_b0, in_b1, out_b0, out_b1)
        pltpu.async_copy(out_b0, out_hbm.at[w, c0 + 2], so_b)
        pltpu.async_copy(out_b1, out_hbm.at[w, c0 + 3], so_b)
        return carry

    lax.fori_loop(0, _NBLK, _loop, 0)

    pltpu.make_async_copy(out_a0, out_hbm.at[w, 0], so_a).wait()
    pltpu.make_async_copy(out_a1, out_hbm.at[w, 1], so_a).wait()
    pltpu.make_async_copy(out_b0, out_hbm.at[w, 2], so_b).wait()
    pltpu.make_async_copy(out_b1, out_hbm.at[w, 3], so_b).wait()


@jax.jit
def kernel(input, grid):
    mesh = plsc.VectorSubcoreMesh(core_axis_name="c", subcore_axis_name="s")
    f = pl.kernel(
        _sc_body,
        mesh=mesh,
        out_type=jax.ShapeDtypeStruct((_N, _C, _L), jnp.float32),
        compiler_params=pltpu.CompilerParams(needs_layout_passes=False),
        scratch_types=[
            pltpu.VMEM((_L,), jnp.float32),
            pltpu.VMEM((_L,), jnp.int32),
            pltpu.VMEM((_L,), jnp.float32),
            pltpu.VMEM((_L,), jnp.float32),
            pltpu.VMEM((_L,), jnp.float32),
            pltpu.VMEM((_L,), jnp.float32),
            pltpu.VMEM((_L,), jnp.float32),
            pltpu.VMEM((_L,), jnp.float32),
            pltpu.VMEM((_L,), jnp.float32),
            pltpu.VMEM((_L,), jnp.float32),
            pltpu.VMEM((_L,), jnp.float32),
            pltpu.SemaphoreType.DMA,
            pltpu.SemaphoreType.DMA,
            pltpu.SemaphoreType.DMA,
            pltpu.SemaphoreType.DMA,
        ],
    )
    return f(input, grid)
